# baseline (speedup 1.0000x reference)
import jax
import jax.numpy as jnp
from jax.experimental import pallas as pl
from jax.experimental.pallas import tpu as pltpu

_BM = 512


def _body(a0, a1, emb_ref, o0, o1):
    o0[...] = jnp.dot(a0[...], emb_ref[...], preferred_element_type=jnp.float32)
    o1[...] = jnp.dot(a1[...], emb_ref[...], preferred_element_type=jnp.float32)


def kernel(adj, embeds):
    M, K = adj.shape
    _, N = embeds.shape
    half_blocks = M // _BM // 2
    out = pl.pallas_call(
        _body,
        grid=(half_blocks,),
        in_specs=[
            pl.BlockSpec((_BM, K), lambda i: (i, 0)),
            pl.BlockSpec((_BM, K), lambda i: (i + 4, 0)),
            pl.BlockSpec((K, N), lambda i: (0, 0)),
        ],
        out_specs=[
            pl.BlockSpec((_BM, N), lambda i: (i, 0)),
            pl.BlockSpec((_BM, N), lambda i: (i, 0)),
        ],
        out_shape=[
            jax.ShapeDtypeStruct((M // 2, N), jnp.float32),
            jax.ShapeDtypeStruct((M // 2, N), jnp.float32),
        ],
        compiler_params=pltpu.CompilerParams(
            dimension_semantics=("parallel",),
        ),
    )(adj, adj, embeds)
    return jnp.concatenate(out, axis=0)


# speedup vs baseline: 1.0164x; 1.0164x over previous
import jax
import jax.numpy as jnp
from jax.experimental import pallas as pl
from jax.experimental.pallas import tpu as pltpu

_BM = 512
_NBUF = 4


def _body(adj_hbm, emb_ref, out_ref, buf, sems):
    i = pl.program_id(0)
    nsteps = pl.num_programs(0)

    def start_copy(slot, blk):
        pltpu.make_async_copy(
            adj_hbm.at[pl.ds(blk * _BM, _BM), :],
            buf.at[slot],
            sems.at[slot],
        ).start()

    @pl.when(i == 0)
    def _warmup():
        for s in range(_NBUF - 1):
            start_copy(s, s)

    nxt = i + _NBUF - 1

    @pl.when(nxt < nsteps)
    def _prefetch():
        start_copy(nxt % _NBUF, nxt)

    slot = i % _NBUF
    pltpu.make_async_copy(
        adj_hbm.at[pl.ds(i * _BM, _BM), :], buf.at[slot], sems.at[slot]
    ).wait()
    out_ref[...] = jnp.dot(buf[slot], emb_ref[...],
                           preferred_element_type=jnp.float32)


def kernel(adj, embeds):
    M, K = adj.shape
    _, N = embeds.shape
    return pl.pallas_call(
        _body,
        grid=(M // _BM,),
        in_specs=[
            pl.BlockSpec(memory_space=pl.ANY),
            pl.BlockSpec((K, N), lambda i: (0, 0)),
        ],
        out_specs=pl.BlockSpec((_BM, N), lambda i: (i, 0)),
        out_shape=jax.ShapeDtypeStruct((M, N), jnp.float32),
        scratch_shapes=[
            pltpu.VMEM((_NBUF, _BM, K), jnp.float32),
            pltpu.SemaphoreType.DMA((_NBUF,)),
        ],
        compiler_params=pltpu.CompilerParams(
            dimension_semantics=("arbitrary",),
        ),
    )(adj, embeds)


# speedup vs baseline: 1.0217x; 1.0053x over previous
import jax
import jax.numpy as jnp
from jax.experimental import pallas as pl
from jax.experimental.pallas import tpu as pltpu

_BM = 256
_NBUF = 8


def _body(adj_hbm, emb_ref, out_ref, buf, sems):
    i = pl.program_id(0)
    nsteps = pl.num_programs(0)

    def start_copy(slot, blk):
        pltpu.make_async_copy(
            adj_hbm.at[pl.ds(blk * _BM, _BM), :],
            buf.at[slot],
            sems.at[slot],
        ).start()

    @pl.when(i == 0)
    def _warmup():
        for s in range(_NBUF - 1):
            start_copy(s, s)

    nxt = i + _NBUF - 1

    @pl.when(nxt < nsteps)
    def _prefetch():
        start_copy(nxt % _NBUF, nxt)

    slot = i % _NBUF
    pltpu.make_async_copy(
        adj_hbm.at[pl.ds(i * _BM, _BM), :], buf.at[slot], sems.at[slot]
    ).wait()
    out_ref[...] = jnp.dot(buf[slot], emb_ref[...],
                           preferred_element_type=jnp.float32)


def kernel(adj, embeds):
    M, K = adj.shape
    _, N = embeds.shape
    return pl.pallas_call(
        _body,
        grid=(M // _BM,),
        in_specs=[
            pl.BlockSpec(memory_space=pl.ANY),
            pl.BlockSpec((K, N), lambda i: (0, 0)),
        ],
        out_specs=pl.BlockSpec((_BM, N), lambda i: (i, 0)),
        out_shape=jax.ShapeDtypeStruct((M, N), jnp.float32),
        scratch_shapes=[
            pltpu.VMEM((_NBUF, _BM, K), jnp.float32),
            pltpu.SemaphoreType.DMA((_NBUF,)),
        ],
        compiler_params=pltpu.CompilerParams(
            dimension_semantics=("arbitrary",),
        ),
    )(adj, embeds)
